# Initial kernel scaffold; baseline (speedup 1.0000x reference)
import jax
import jax.numpy as jnp
from jax.experimental import pallas as pl


def kernel(x, edge_index, W1, a_src1, a_dst1, b1, W2, a_src2, a_dst2, b2):
    raise NotImplementedError("write your pallas kernel here")


# speedup vs baseline: 3.0764x; 3.0764x over previous
import functools

import jax
import jax.numpy as jnp
import numpy as np
from jax.experimental import pallas as pl
from jax.experimental.pallas import tpu as pltpu

N = 10000
E = 320000
D = 128
H1 = 8
C1 = 8
NC = 40


def _prep_body(x_ref, w_ref, asrc_ref, adst_ref, h_ref, as_ref, ad_ref):
    h = jnp.dot(x_ref[...], w_ref[...], preferred_element_type=jnp.float32)
    h_ref[...] = h
    as_ref[...] = jnp.dot(h, asrc_ref[...], preferred_element_type=jnp.float32)
    ad_ref[...] = jnp.dot(h, adst_ref[...], preferred_element_type=jnp.float32)


def _prep(x, W, A_src, A_dst, H):
    n, d = x.shape
    f = W.shape[1]
    blocks = 5
    rb = n // blocks
    return pl.pallas_call(
        _prep_body,
        grid=(blocks,),
        in_specs=[
            pl.BlockSpec((rb, d), lambda i: (i, 0)),
            pl.BlockSpec((d, f), lambda i: (0, 0)),
            pl.BlockSpec((f, H), lambda i: (0, 0)),
            pl.BlockSpec((f, H), lambda i: (0, 0)),
        ],
        out_specs=[
            pl.BlockSpec((rb, f), lambda i: (i, 0)),
            pl.BlockSpec((rb, H), lambda i: (i, 0)),
            pl.BlockSpec((rb, H), lambda i: (i, 0)),
        ],
        out_shape=[
            jax.ShapeDtypeStruct((n, f), jnp.float32),
            jax.ShapeDtypeStruct((n, H), jnp.float32),
            jax.ShapeDtypeStruct((n, H), jnp.float32),
        ],
    )(x, W, A_src, A_dst)


def _edge_phase(h, alpha_s, alpha_d, src, dst, H, C, n):
    alpha = alpha_s[src] + alpha_d[dst]
    alpha = jax.nn.leaky_relu(alpha, 0.2)
    amax = jax.ops.segment_max(alpha, dst, num_segments=n)
    ex = jnp.exp(alpha - amax[dst])
    denom = jax.ops.segment_sum(ex, dst, num_segments=n)
    msg = h[src].reshape(-1, H, C) * (ex / (denom[dst] + 1e-16))[:, :, None]
    out = jax.ops.segment_sum(msg.reshape(-1, H * C), dst, num_segments=n)
    return out


def _expand_a(a, H, C):
    m = np.zeros((H * C, H), dtype=np.float32)
    out = jnp.zeros((H * C, H), jnp.float32)
    idx = jnp.arange(H * C)
    out = out.at[idx, idx // C].set(a.reshape(-1))
    return out


def kernel(x, edge_index, W1, a_src1, a_dst1, b1, W2, a_src2, a_dst2, b2):
    n = x.shape[0]
    loop = jnp.arange(n, dtype=edge_index.dtype)
    src = jnp.concatenate([edge_index[0], loop])
    dst = jnp.concatenate([edge_index[1], loop])

    A_src1 = _expand_a(a_src1, H1, C1)
    A_dst1 = _expand_a(a_dst1, H1, C1)
    h1, as1, ad1 = _prep(x, W1, A_src1, A_dst1, H1)
    o1 = _edge_phase(h1, as1, ad1, src, dst, H1, C1, n)
    hmid = jax.nn.elu(o1 + b1)

    A_src2 = _expand_a(a_src2, 1, NC)
    A_dst2 = _expand_a(a_dst2, 1, NC)
    h2, as2, ad2 = _prep(hmid, W2, A_src2, A_dst2, 1)
    o2 = _edge_phase(h2, as2, ad2, src, dst, 1, NC, n)
    out = o2 + b2
    return jax.nn.log_softmax(out, axis=-1)


# speedup vs baseline: 15.1638x; 4.9291x over previous
import functools

import jax
import jax.numpy as jnp
import numpy as np
from jax import lax
from jax.experimental import pallas as pl
from jax.experimental.pallas import tpu as pltpu
from jax.experimental.pallas import tpu_sc as plsc

N = 10000
E = 320000
D = 128
H1 = 8
C1 = 8
NC = 40

NCORE = 2
NSUB = 16
NW = NCORE * NSUB

CHUNK = 64
EROWS = E // CHUNK

SW1 = 128
DW1 = 64
AW1 = 128
SW2 = 96
DW2 = 48
AW2 = 48

RB = 2000
ZR = 208
NPS = 624

_SC_MESH = plsc.VectorSubcoreMesh(
    core_axis_name="c", subcore_axis_name="s",
    num_cores=NCORE, num_subcores=NSUB)
_SC_PARAMS = pltpu.CompilerParams(use_tc_tiling_on_sc=False)


def _prep1_body(x_ref, w_ref, asrc_ref, adst_ref, stab_ref, adtab_ref, m_ref):
    i = pl.program_id(0)
    h = jnp.dot(x_ref[...], w_ref[...], preferred_element_type=jnp.float32)
    a_s = jnp.dot(h, asrc_ref[...], preferred_element_type=jnp.float32)
    a_d = jnp.dot(h, adst_ref[...], preferred_element_type=jnp.float32)
    stab_ref[...] = jnp.concatenate([h, a_s], axis=1)
    adtab_ref[...] = a_d
    bm = jnp.concatenate(
        [jnp.max(a_s, axis=0, keepdims=True),
         jnp.max(a_d, axis=0, keepdims=True)], axis=1)

    @pl.when(i == 0)
    def _():
        m_ref[...] = bm

    @pl.when(i > 0)
    def _():
        m_ref[...] = jnp.maximum(m_ref[...], bm)


def _prep1(x, W1, Asrc_exp, Adst_exp):
    return pl.pallas_call(
        _prep1_body,
        grid=(N // RB,),
        in_specs=[
            pl.BlockSpec((RB, D), lambda i: (i, 0)),
            pl.BlockSpec((D, H1 * C1), lambda i: (0, 0)),
            pl.BlockSpec((H1 * C1, 64), lambda i: (0, 0)),
            pl.BlockSpec((H1 * C1, 64), lambda i: (0, 0)),
        ],
        out_specs=[
            pl.BlockSpec((RB, SW1), lambda i: (i, 0)),
            pl.BlockSpec((RB, DW1), lambda i: (i, 0)),
            pl.BlockSpec((1, 128), lambda i: (0, 0)),
        ],
        out_shape=[
            jax.ShapeDtypeStruct((N, SW1), jnp.float32),
            jax.ShapeDtypeStruct((N, DW1), jnp.float32),
            jax.ShapeDtypeStruct((1, 128), jnp.float32),
        ],
    )(x, W1, Asrc_exp, Adst_exp)


def _zero_acc(zbuf, acc, sid, rw):
    def _zf(r, c):
        for j in range(rw // 16):
            zbuf[r, pl.ds(j * 16, 16)] = jnp.zeros((16,), jnp.float32)
        return c
    lax.fori_loop(0, ZR, _zf, 0)
    nbase = sid * NPS
    for k in range(3):
        pltpu.sync_copy(zbuf, acc.at[pl.ds(nbase + k * ZR, ZR)])

    @pl.when(sid == NSUB - 1)
    def _():
        pltpu.sync_copy(zbuf.at[pl.ds(0, 16)], acc.at[pl.ds(NSUB * NPS, 16)])


def _publish_acc(acc, out, cid, sid):
    nbase = sid * NPS
    pltpu.sync_copy(acc.at[pl.ds(nbase, NPS)], out.at[pl.ds(nbase, NPS)])

    @pl.when(sid == NSUB - 1)
    def _():
        pltpu.sync_copy(acc.at[pl.ds(NSUB * NPS, 16)],
                        out.at[pl.ds(NSUB * NPS, 16)])


def _edge_pass_body(hoff, nj, stab, adtab, sidx, didx, mexp, out,
                    idx_v, didx_v, srows, drows, msg, mexp_v, zbuf,
                    acc, sem1, sem2, sem3):
    cid = lax.axis_index("c")
    sid = lax.axis_index("s")

    @pl.when(cid == 0)
    def _():
        _zero_acc(zbuf, acc, sid, nj * 16)
        pltpu.sync_copy(mexp, mexp_v)
        plsc.subcore_barrier()

        mvs = [mexp_v[pl.ds(j * 16, 16)] for j in range(nj)]
        nr = (EROWS - sid + NSUB - 1) // NSUB

        def _row(r, c):
            row = sid + r * NSUB
            pltpu.sync_copy(sidx.at[row], idx_v)
            pltpu.sync_copy(didx.at[row], didx_v)
            d1 = pltpu.make_async_copy(stab.at[idx_v], srows, sem1)
            d2 = pltpu.make_async_copy(adtab.at[didx_v], drows, sem2)
            d1.start()
            d2.start()
            d1.wait()
            d2.wait()

            def _edge(ee, c2):
                for j in range(nj):
                    a = srows[ee, pl.ds(hoff + j * 16, 16)]
                    b = drows[ee, pl.ds(j * 16, 16)]
                    t = a + b
                    t = jnp.maximum(t, 0.2 * t) - mvs[j]
                    ex = jnp.exp(t)
                    hv = srows[ee, pl.ds(j * 16, 16)]
                    msg[ee, pl.ds(j * 16, 16)] = hv * ex
                    if hoff == 64:
                        msg[ee, pl.ds(64 + j * 16, 16)] = ex
                return c2

            lax.fori_loop(0, CHUNK, _edge, 0)
            d3 = pltpu.make_async_copy(msg, acc.at[didx_v], sem3)
            d3.start(add=True)
            d3.wait()
            return c

        lax.fori_loop(0, nr, _row, 0)

        plsc.subcore_barrier()
        _publish_acc(acc, out, cid, sid)


def _make_edge_pass(hoff, nj, sw, dw, aw):
    body = functools.partial(_edge_pass_body, hoff, nj)
    return functools.partial(
        pl.kernel,
        out_type=jax.ShapeDtypeStruct((N, aw), jnp.float32),
        mesh=_SC_MESH,
        compiler_params=_SC_PARAMS,
        scratch_types=[
            pltpu.VMEM((CHUNK,), jnp.int32),
            pltpu.VMEM((CHUNK,), jnp.int32),
            pltpu.VMEM((CHUNK, sw), jnp.float32),
            pltpu.VMEM((CHUNK, dw), jnp.float32),
            pltpu.VMEM((CHUNK, aw), jnp.float32),
            pltpu.VMEM((nj * 16,), jnp.float32),
            pltpu.VMEM((ZR, aw), jnp.float32),
            pltpu.VMEM_SHARED((N, aw), jnp.float32),
            pltpu.SemaphoreType.DMA,
            pltpu.SemaphoreType.DMA,
            pltpu.SemaphoreType.DMA,
        ],
    )(body)


_sc1 = _make_edge_pass(64, 4, SW1, DW1, AW1)
_sc2 = _make_edge_pass(48, 3, SW2, DW2, AW2)


def _mid_body(acc_ref, stab_ref, adtab_ref, mexp_ref, b1_ref, w2_ref,
              a2s_ref, a2d_ref, stab2_ref, adtab2_ref, m2_ref):
    i = pl.program_id(0)
    acc = acc_ref[...]
    h1 = stab_ref[:, 0:64]
    as_e = stab_ref[:, 64:128]
    ad_e = adtab_ref[...]
    t = as_e + ad_e
    exs = jnp.exp(jnp.maximum(t, 0.2 * t) - mexp_ref[...])
    num = acc[:, 0:64] + exs * h1
    den = acc[:, 64:128] + exs + 1e-16
    o1 = num / den + b1_ref[...]
    hmid = jnp.where(o1 > 0, o1, jnp.exp(jnp.minimum(o1, 0.0)) - 1.0)
    h2 = jnp.dot(hmid, w2_ref[...], preferred_element_type=jnp.float32)
    as2 = jnp.dot(h2, a2s_ref[...], preferred_element_type=jnp.float32)
    ad2 = jnp.dot(h2, a2d_ref[...], preferred_element_type=jnp.float32)
    ones = jnp.ones((RB, 1), jnp.float32)
    stab2_ref[...] = jnp.concatenate(
        [h2, ones, jnp.zeros((RB, 7), jnp.float32),
         jnp.broadcast_to(as2, (RB, 48))], axis=1)
    adtab2_ref[...] = jnp.broadcast_to(ad2, (RB, DW2))
    bm = jnp.concatenate(
        [jnp.max(as2, axis=0, keepdims=True),
         jnp.max(ad2, axis=0, keepdims=True),
         jnp.zeros((1, 14), jnp.float32)], axis=1)

    @pl.when(i == 0)
    def _():
        m2_ref[...] = bm

    @pl.when(i > 0)
    def _():
        m2_ref[...] = jnp.maximum(m2_ref[...], bm)


def _mid(acc1, stab, adtab, mexp1, b1, W2, A2s, A2d):
    return pl.pallas_call(
        _mid_body,
        grid=(N // RB,),
        in_specs=[
            pl.BlockSpec((RB, AW1), lambda i: (i, 0)),
            pl.BlockSpec((RB, SW1), lambda i: (i, 0)),
            pl.BlockSpec((RB, DW1), lambda i: (i, 0)),
            pl.BlockSpec((1, 64), lambda i: (0, 0)),
            pl.BlockSpec((1, 64), lambda i: (0, 0)),
            pl.BlockSpec((64, NC), lambda i: (0, 0)),
            pl.BlockSpec((NC, 1), lambda i: (0, 0)),
            pl.BlockSpec((NC, 1), lambda i: (0, 0)),
        ],
        out_specs=[
            pl.BlockSpec((RB, SW2), lambda i: (i, 0)),
            pl.BlockSpec((RB, DW2), lambda i: (i, 0)),
            pl.BlockSpec((1, 16), lambda i: (0, 0)),
        ],
        out_shape=[
            jax.ShapeDtypeStruct((N, SW2), jnp.float32),
            jax.ShapeDtypeStruct((N, DW2), jnp.float32),
            jax.ShapeDtypeStruct((1, 16), jnp.float32),
        ],
    )(acc1, stab, adtab, mexp1, b1, W2, A2s, A2d)


def _fin_body(acc_ref, stab2_ref, adtab2_ref, m2_ref, b2_ref, out_ref):
    acc = acc_ref[...]
    h2 = stab2_ref[:, 0:40]
    as2 = stab2_ref[:, 48:49]
    ad2 = adtab2_ref[:, 0:1]
    m2 = m2_ref[0, 0] + m2_ref[0, 1]
    m2 = jnp.maximum(m2, 0.2 * m2)
    t = as2 + ad2
    ex = jnp.exp(jnp.maximum(t, 0.2 * t) - m2)
    num = acc[:, 0:40] + ex * h2
    den = acc[:, 40:41] + ex + 1e-16
    o2 = num / den + b2_ref[...]
    mx = jnp.max(o2, axis=1, keepdims=True)
    z = o2 - mx
    lse = jnp.log(jnp.sum(jnp.exp(z), axis=1, keepdims=True))
    out_ref[...] = z - lse


def _fin(acc2, stab2, adtab2, m2, b2):
    return pl.pallas_call(
        _fin_body,
        grid=(N // RB,),
        in_specs=[
            pl.BlockSpec((RB, AW2), lambda i: (i, 0)),
            pl.BlockSpec((RB, SW2), lambda i: (i, 0)),
            pl.BlockSpec((RB, DW2), lambda i: (i, 0)),
            pl.BlockSpec((1, 16), lambda i: (0, 0)),
            pl.BlockSpec((1, NC), lambda i: (0, 0)),
        ],
        out_specs=pl.BlockSpec((RB, NC), lambda i: (i, 0)),
        out_shape=jax.ShapeDtypeStruct((N, NC), jnp.float32),
    )(acc2, stab2, adtab2, m2, b2)


def kernel(x, edge_index, W1, a_src1, a_dst1, b1, W2, a_src2, a_dst2, b2):
    sidx = edge_index[0].astype(jnp.int32).reshape(EROWS, CHUNK)
    didx = edge_index[1].astype(jnp.int32).reshape(EROWS, CHUNK)

    r8 = np.zeros((H1, H1 * C1), np.float32)
    for hh in range(H1):
        r8[hh, hh * C1:(hh + 1) * C1] = 1.0
    R8 = jnp.asarray(r8)
    Asrc_exp = _expand_a(a_src1, H1, C1) @ R8
    Adst_exp = _expand_a(a_dst1, H1, C1) @ R8

    stab, adtab, m1 = _prep1(x, W1, Asrc_exp, Adst_exp)
    ms = m1[0, 0:64] + m1[0, 64:128]
    mexp1 = jnp.maximum(ms, 0.2 * ms)

    acc1 = _sc1(stab, adtab, sidx, didx, mexp1)

    stab2, adtab2, m2 = _mid(acc1, stab, adtab, mexp1.reshape(1, 64),
                             b1.reshape(1, -1), W2,
                             a_src2.reshape(-1, 1), a_dst2.reshape(-1, 1))
    s2 = m2[0, 0] + m2[0, 1]
    M2 = jnp.maximum(s2, 0.2 * s2)
    mexp2 = jnp.full((48,), M2, jnp.float32)

    acc2 = _sc2(stab2, adtab2, sidx, didx, mexp2)

    return _fin(acc2, stab2, adtab2, m2, b2.reshape(1, -1))


def _expand_a(a, H, C):
    out = jnp.zeros((H * C, H), jnp.float32)
    idx = jnp.arange(H * C)
    return out.at[idx, idx // C].set(a.reshape(-1))


# speedup vs baseline: 16.9752x; 1.1195x over previous
import functools

import jax
import jax.numpy as jnp
import numpy as np
from jax import lax
from jax.experimental import pallas as pl
from jax.experimental.pallas import tpu as pltpu
from jax.experimental.pallas import tpu_sc as plsc

N = 10000
E = 320000
D = 128
H1 = 8
C1 = 8
NC = 40

NCORE = 2
NSUB = 16
NW = NCORE * NSUB

CHUNK1 = 32
CHUNK2 = 64
EROWS1 = E // CHUNK1
EROWS2 = E // CHUNK2

SW1 = 128
DW1 = 64
AW1 = 128
SW2 = 96
DW2 = 48
AW2 = 48

RB = 2000
ZR = 208
NPS = 624

_SC_MESH = plsc.VectorSubcoreMesh(
    core_axis_name="c", subcore_axis_name="s",
    num_cores=NCORE, num_subcores=NSUB)
_SC_PARAMS = pltpu.CompilerParams(use_tc_tiling_on_sc=False)


def _prep1_body(x_ref, w_ref, asrc_ref, adst_ref, stab_ref, adtab_ref, m_ref):
    i = pl.program_id(0)
    h = jnp.dot(x_ref[...], w_ref[...], preferred_element_type=jnp.float32)
    a_s = jnp.dot(h, asrc_ref[...], preferred_element_type=jnp.float32)
    a_d = jnp.dot(h, adst_ref[...], preferred_element_type=jnp.float32)
    stab_ref[...] = jnp.concatenate([h, a_s], axis=1)
    adtab_ref[...] = a_d
    bm = jnp.concatenate(
        [jnp.max(a_s, axis=0, keepdims=True),
         jnp.max(a_d, axis=0, keepdims=True)], axis=1)

    @pl.when(i == 0)
    def _():
        m_ref[...] = bm

    @pl.when(i > 0)
    def _():
        m_ref[...] = jnp.maximum(m_ref[...], bm)


def _prep1(x, W1, Asrc_exp, Adst_exp):
    return pl.pallas_call(
        _prep1_body,
        grid=(N // RB,),
        in_specs=[
            pl.BlockSpec((RB, D), lambda i: (i, 0)),
            pl.BlockSpec((D, H1 * C1), lambda i: (0, 0)),
            pl.BlockSpec((H1 * C1, 64), lambda i: (0, 0)),
            pl.BlockSpec((H1 * C1, 64), lambda i: (0, 0)),
        ],
        out_specs=[
            pl.BlockSpec((RB, SW1), lambda i: (i, 0)),
            pl.BlockSpec((RB, DW1), lambda i: (i, 0)),
            pl.BlockSpec((1, 128), lambda i: (0, 0)),
        ],
        out_shape=[
            jax.ShapeDtypeStruct((N, SW1), jnp.float32),
            jax.ShapeDtypeStruct((N, DW1), jnp.float32),
            jax.ShapeDtypeStruct((1, 128), jnp.float32),
        ],
    )(x, W1, Asrc_exp, Adst_exp)


def _zero_acc(zbuf, acc, sid, rw):
    def _zf(r, c):
        for j in range(rw // 16):
            zbuf[r, pl.ds(j * 16, 16)] = jnp.zeros((16,), jnp.float32)
        return c
    lax.fori_loop(0, ZR, _zf, 0)
    nbase = sid * NPS
    for k in range(3):
        pltpu.sync_copy(zbuf, acc.at[pl.ds(nbase + k * ZR, ZR)])

    @pl.when(sid == NSUB - 1)
    def _():
        pltpu.sync_copy(zbuf.at[pl.ds(0, 16)], acc.at[pl.ds(NSUB * NPS, 16)])


def _publish_acc(acc, out, cid, sid):
    nbase = sid * NPS
    pltpu.sync_copy(acc.at[pl.ds(nbase, NPS)], out.at[pl.ds(nbase, NPS)])

    @pl.when(sid == NSUB - 1)
    def _():
        pltpu.sync_copy(acc.at[pl.ds(NSUB * NPS, 16)],
                        out.at[pl.ds(NSUB * NPS, 16)])


def _edge_pass_body(hoff, nj, chunk, npairs, mk_start, tail_cond,
                    stab, adtab, sidx, didx, mexp, out,
                    sidx2, didxr, sra, dra, srb, drb, msga, msgb, mexp_v,
                    zbuf, acc, sas, sad, sbs, sbd, sca, scb):
    cid = lax.axis_index("c")
    sid = lax.axis_index("s")

    @pl.when(cid == 0)
    def _():
        _zero_acc(zbuf, acc, sid, nj * 16)
        pltpu.sync_copy(mexp, mexp_v)
        plsc.subcore_barrier()

        mvs = [mexp_v[pl.ds(j * 16, 16)] for j in range(nj)]
        start = mk_start(sid)

        def _compute(srows, drows, msg):
            def _edge(ee, c2):
                for j in range(nj):
                    a = srows[ee, pl.ds(hoff + j * 16, 16)]
                    b = drows[ee, pl.ds(j * 16, 16)]
                    t = a + b
                    t = jnp.maximum(t, 0.2 * t) - mvs[j]
                    ex = jnp.exp(t)
                    hv = srows[ee, pl.ds(j * 16, 16)]
                    msg[ee, pl.ds(j * 16, 16)] = hv * ex
                    if hoff == 64:
                        msg[ee, pl.ds(64 + j * 16, 16)] = ex
                return c2

            lax.fori_loop(0, chunk, _edge, 0)

        def _pair(g, c):
            row = start + 2 * g
            base = (g & 1) * 2
            pltpu.sync_copy(sidx.at[pl.ds(row, 2)], sidx2)
            pltpu.sync_copy(didx.at[pl.ds(row, 2)], didxr.at[pl.ds(base, 2)])
            ga_s = pltpu.make_async_copy(stab.at[sidx2.at[0]], sra, sas)
            ga_d = pltpu.make_async_copy(adtab.at[didxr.at[base]], dra, sad)
            gb_s = pltpu.make_async_copy(stab.at[sidx2.at[1]], srb, sbs)
            gb_d = pltpu.make_async_copy(adtab.at[didxr.at[base + 1]], drb, sbd)
            ga_s.start()
            ga_d.start()
            gb_s.start()
            gb_d.start()
            ga_s.wait()
            ga_d.wait()

            @pl.when(g > 0)
            def _():
                pltpu.make_async_copy(msga, acc.at[didxr.at[base]], sca).wait()

            _compute(sra, dra, msga)
            da = pltpu.make_async_copy(msga, acc.at[didxr.at[base]], sca)
            da.start(add=True)
            gb_s.wait()
            gb_d.wait()

            @pl.when(g > 0)
            def _():
                pltpu.make_async_copy(
                    msgb, acc.at[didxr.at[base + 1]], scb).wait()

            _compute(srb, drb, msgb)
            db = pltpu.make_async_copy(msgb, acc.at[didxr.at[base + 1]], scb)
            db.start(add=True)
            return c

        lax.fori_loop(0, npairs, _pair, 0)
        pltpu.make_async_copy(msga, acc.at[didxr.at[0]], sca).wait()
        pltpu.make_async_copy(msgb, acc.at[didxr.at[1]], scb).wait()

        @pl.when(tail_cond(sid))
        def _tail():
            row = start + 2 * npairs
            pltpu.sync_copy(sidx.at[row], sidx2.at[0])
            pltpu.sync_copy(didx.at[row], didxr.at[0])
            ga_s = pltpu.make_async_copy(stab.at[sidx2.at[0]], sra, sas)
            ga_d = pltpu.make_async_copy(adtab.at[didxr.at[0]], dra, sad)
            ga_s.start()
            ga_d.start()
            ga_s.wait()
            ga_d.wait()
            _compute(sra, dra, msga)
            da = pltpu.make_async_copy(msga, acc.at[didxr.at[0]], sca)
            da.start(add=True)
            da.wait()

        plsc.subcore_barrier()
        _publish_acc(acc, out, cid, sid)


def _make_edge_pass(hoff, nj, chunk, npairs, mk_start, tail_cond, sw, dw, aw):
    body = functools.partial(_edge_pass_body, hoff, nj, chunk, npairs,
                             mk_start, tail_cond)
    return functools.partial(
        pl.kernel,
        out_type=jax.ShapeDtypeStruct((N, aw), jnp.float32),
        mesh=_SC_MESH,
        compiler_params=_SC_PARAMS,
        scratch_types=[
            pltpu.VMEM((2, chunk), jnp.int32),
            pltpu.VMEM((4, chunk), jnp.int32),
            pltpu.VMEM((chunk, sw), jnp.float32),
            pltpu.VMEM((chunk, dw), jnp.float32),
            pltpu.VMEM((chunk, sw), jnp.float32),
            pltpu.VMEM((chunk, dw), jnp.float32),
            pltpu.VMEM((chunk, aw), jnp.float32),
            pltpu.VMEM((chunk, aw), jnp.float32),
            pltpu.VMEM((nj * 16,), jnp.float32),
            pltpu.VMEM((ZR, aw), jnp.float32),
            pltpu.VMEM_SHARED((N, aw), jnp.float32),
            pltpu.SemaphoreType.DMA,
            pltpu.SemaphoreType.DMA,
            pltpu.SemaphoreType.DMA,
            pltpu.SemaphoreType.DMA,
            pltpu.SemaphoreType.DMA,
            pltpu.SemaphoreType.DMA,
        ],
    )(body)


_sc1 = _make_edge_pass(
    64, 4, CHUNK1, (EROWS1 // NSUB) // 2,
    lambda sid: sid * (EROWS1 // NSUB), lambda sid: sid >= 0,
    SW1, DW1, AW1)
_sc2 = _make_edge_pass(
    48, 3, CHUNK2, 156,
    lambda sid: sid * 313 - jnp.maximum(sid - 8, 0), lambda sid: sid < 8,
    SW2, DW2, AW2)


def _mid_body(acc_ref, stab_ref, adtab_ref, mexp_ref, b1_ref, w2_ref,
              a2s_ref, a2d_ref, stab2_ref, adtab2_ref, m2_ref):
    i = pl.program_id(0)
    acc = acc_ref[...]
    h1 = stab_ref[:, 0:64]
    as_e = stab_ref[:, 64:128]
    ad_e = adtab_ref[...]
    t = as_e + ad_e
    exs = jnp.exp(jnp.maximum(t, 0.2 * t) - mexp_ref[...])
    num = acc[:, 0:64] + exs * h1
    den = acc[:, 64:128] + exs + 1e-16
    o1 = num / den + b1_ref[...]
    hmid = jnp.where(o1 > 0, o1, jnp.exp(jnp.minimum(o1, 0.0)) - 1.0)
    h2 = jnp.dot(hmid, w2_ref[...], preferred_element_type=jnp.float32)
    as2 = jnp.dot(h2, a2s_ref[...], preferred_element_type=jnp.float32)
    ad2 = jnp.dot(h2, a2d_ref[...], preferred_element_type=jnp.float32)
    ones = jnp.ones((RB, 1), jnp.float32)
    stab2_ref[...] = jnp.concatenate(
        [h2, ones, jnp.zeros((RB, 7), jnp.float32),
         jnp.broadcast_to(as2, (RB, 48))], axis=1)
    adtab2_ref[...] = jnp.broadcast_to(ad2, (RB, DW2))
    bm = jnp.concatenate(
        [jnp.max(as2, axis=0, keepdims=True),
         jnp.max(ad2, axis=0, keepdims=True),
         jnp.zeros((1, 14), jnp.float32)], axis=1)

    @pl.when(i == 0)
    def _():
        m2_ref[...] = bm

    @pl.when(i > 0)
    def _():
        m2_ref[...] = jnp.maximum(m2_ref[...], bm)


def _mid(acc1, stab, adtab, mexp1, b1, W2, A2s, A2d):
    return pl.pallas_call(
        _mid_body,
        grid=(N // RB,),
        in_specs=[
            pl.BlockSpec((RB, AW1), lambda i: (i, 0)),
            pl.BlockSpec((RB, SW1), lambda i: (i, 0)),
            pl.BlockSpec((RB, DW1), lambda i: (i, 0)),
            pl.BlockSpec((1, 64), lambda i: (0, 0)),
            pl.BlockSpec((1, 64), lambda i: (0, 0)),
            pl.BlockSpec((64, NC), lambda i: (0, 0)),
            pl.BlockSpec((NC, 1), lambda i: (0, 0)),
            pl.BlockSpec((NC, 1), lambda i: (0, 0)),
        ],
        out_specs=[
            pl.BlockSpec((RB, SW2), lambda i: (i, 0)),
            pl.BlockSpec((RB, DW2), lambda i: (i, 0)),
            pl.BlockSpec((1, 16), lambda i: (0, 0)),
        ],
        out_shape=[
            jax.ShapeDtypeStruct((N, SW2), jnp.float32),
            jax.ShapeDtypeStruct((N, DW2), jnp.float32),
            jax.ShapeDtypeStruct((1, 16), jnp.float32),
        ],
    )(acc1, stab, adtab, mexp1, b1, W2, A2s, A2d)


def _fin_body(acc_ref, stab2_ref, adtab2_ref, m2_ref, b2_ref, out_ref):
    acc = acc_ref[...]
    h2 = stab2_ref[:, 0:40]
    as2 = stab2_ref[:, 48:49]
    ad2 = adtab2_ref[:, 0:1]
    m2 = m2_ref[0, 0] + m2_ref[0, 1]
    m2 = jnp.maximum(m2, 0.2 * m2)
    t = as2 + ad2
    ex = jnp.exp(jnp.maximum(t, 0.2 * t) - m2)
    num = acc[:, 0:40] + ex * h2
    den = acc[:, 40:41] + ex + 1e-16
    o2 = num / den + b2_ref[...]
    mx = jnp.max(o2, axis=1, keepdims=True)
    z = o2 - mx
    lse = jnp.log(jnp.sum(jnp.exp(z), axis=1, keepdims=True))
    out_ref[...] = z - lse


def _fin(acc2, stab2, adtab2, m2, b2):
    return pl.pallas_call(
        _fin_body,
        grid=(N // RB,),
        in_specs=[
            pl.BlockSpec((RB, AW2), lambda i: (i, 0)),
            pl.BlockSpec((RB, SW2), lambda i: (i, 0)),
            pl.BlockSpec((RB, DW2), lambda i: (i, 0)),
            pl.BlockSpec((1, 16), lambda i: (0, 0)),
            pl.BlockSpec((1, NC), lambda i: (0, 0)),
        ],
        out_specs=pl.BlockSpec((RB, NC), lambda i: (i, 0)),
        out_shape=jax.ShapeDtypeStruct((N, NC), jnp.float32),
    )(acc2, stab2, adtab2, m2, b2)


def kernel(x, edge_index, W1, a_src1, a_dst1, b1, W2, a_src2, a_dst2, b2):
    src_i = edge_index[0].astype(jnp.int32)
    dst_i = edge_index[1].astype(jnp.int32)
    sidx1 = src_i.reshape(EROWS1, CHUNK1)
    didx1 = dst_i.reshape(EROWS1, CHUNK1)
    sidx2v = src_i.reshape(EROWS2, CHUNK2)
    didx2v = dst_i.reshape(EROWS2, CHUNK2)

    r8 = np.zeros((H1, H1 * C1), np.float32)
    for hh in range(H1):
        r8[hh, hh * C1:(hh + 1) * C1] = 1.0
    R8 = jnp.asarray(r8)
    Asrc_exp = _expand_a(a_src1, H1, C1) @ R8
    Adst_exp = _expand_a(a_dst1, H1, C1) @ R8

    stab, adtab, m1 = _prep1(x, W1, Asrc_exp, Adst_exp)
    ms = m1[0, 0:64] + m1[0, 64:128]
    mexp1 = jnp.maximum(ms, 0.2 * ms)

    acc1 = _sc1(stab, adtab, sidx1, didx1, mexp1)

    stab2, adtab2, m2 = _mid(acc1, stab, adtab, mexp1.reshape(1, 64),
                             b1.reshape(1, -1), W2,
                             a_src2.reshape(-1, 1), a_dst2.reshape(-1, 1))
    s2 = m2[0, 0] + m2[0, 1]
    M2 = jnp.maximum(s2, 0.2 * s2)
    mexp2 = jnp.full((48,), M2, jnp.float32)

    acc2 = _sc2(stab2, adtab2, sidx2v, didx2v, mexp2)

    return _fin(acc2, stab2, adtab2, m2, b2.reshape(1, -1))


def _expand_a(a, H, C):
    out = jnp.zeros((H * C, H), jnp.float32)
    idx = jnp.arange(H * C)
    return out.at[idx, idx // C].set(a.reshape(-1))


# speedup vs baseline: 17.5771x; 1.0355x over previous
import functools

import jax
import jax.numpy as jnp
import numpy as np
from jax import lax
from jax.experimental import pallas as pl
from jax.experimental.pallas import tpu as pltpu
from jax.experimental.pallas import tpu_sc as plsc

N = 10000
E = 320000
D = 128
H1 = 8
C1 = 8
NC = 40

NCORE = 2
NSUB = 16
NW = NCORE * NSUB

CHUNK1 = 32
CHUNK2 = 64
EROWS1 = E // CHUNK1
EROWS2 = E // CHUNK2

SW1 = 128
DW1 = 64
AW1 = 128
SW2 = 96
DW2 = 48
AW2 = 48

RB = 2000
ZR = 208
NPS = 624

_SC_MESH = plsc.VectorSubcoreMesh(
    core_axis_name="c", subcore_axis_name="s",
    num_cores=NCORE, num_subcores=NSUB)
_SC_PARAMS = pltpu.CompilerParams(use_tc_tiling_on_sc=False)


def _prep1_body(x_ref, w_ref, asrc_ref, adst_ref, stab_ref, adtab_ref, m_ref):
    i = pl.program_id(0)
    h = jnp.dot(x_ref[...], w_ref[...], preferred_element_type=jnp.float32)
    a_s = jnp.dot(h, asrc_ref[...], preferred_element_type=jnp.float32)
    a_d = jnp.dot(h, adst_ref[...], preferred_element_type=jnp.float32)
    stab_ref[...] = jnp.concatenate([h, a_s], axis=1)
    adtab_ref[...] = a_d
    bm = jnp.concatenate(
        [jnp.max(a_s, axis=0, keepdims=True),
         jnp.max(a_d, axis=0, keepdims=True)], axis=1)

    @pl.when(i == 0)
    def _():
        m_ref[...] = bm

    @pl.when(i > 0)
    def _():
        m_ref[...] = jnp.maximum(m_ref[...], bm)


def _prep1(x, W1, Asrc_exp, Adst_exp):
    return pl.pallas_call(
        _prep1_body,
        grid=(N // RB,),
        in_specs=[
            pl.BlockSpec((RB, D), lambda i: (i, 0)),
            pl.BlockSpec((D, H1 * C1), lambda i: (0, 0)),
            pl.BlockSpec((H1 * C1, 64), lambda i: (0, 0)),
            pl.BlockSpec((H1 * C1, 64), lambda i: (0, 0)),
        ],
        out_specs=[
            pl.BlockSpec((RB, SW1), lambda i: (i, 0)),
            pl.BlockSpec((RB, DW1), lambda i: (i, 0)),
            pl.BlockSpec((1, 128), lambda i: (0, 0)),
        ],
        out_shape=[
            jax.ShapeDtypeStruct((N, SW1), jnp.float32),
            jax.ShapeDtypeStruct((N, DW1), jnp.float32),
            jax.ShapeDtypeStruct((1, 128), jnp.float32),
        ],
    )(x, W1, Asrc_exp, Adst_exp)


def _zero_acc(zbuf, acc, sid, rw):
    def _zf(r, c):
        for j in range(rw // 16):
            zbuf[r, pl.ds(j * 16, 16)] = jnp.zeros((16,), jnp.float32)
        return c
    lax.fori_loop(0, ZR, _zf, 0)
    nbase = sid * NPS
    for k in range(3):
        pltpu.sync_copy(zbuf, acc.at[pl.ds(nbase + k * ZR, ZR)])

    @pl.when(sid == NSUB - 1)
    def _():
        pltpu.sync_copy(zbuf.at[pl.ds(0, 16)], acc.at[pl.ds(NSUB * NPS, 16)])


def _publish_acc(acc, out, cid, sid):
    nbase = sid * NPS
    pltpu.sync_copy(acc.at[pl.ds(nbase, NPS)], out.at[pl.ds(nbase, NPS)])

    @pl.when(sid == NSUB - 1)
    def _():
        pltpu.sync_copy(acc.at[pl.ds(NSUB * NPS, 16)],
                        out.at[pl.ds(NSUB * NPS, 16)])


def _edge_pass_body(hoff, nj, chunk, npairs, mk_start, tail_cond,
                    stab, adtab, sidx, didx, mexp, out,
                    sidxa, didxa, sidxb, didxb, sra, dra, srb, drb,
                    msga, msgb, mexp_v,
                    zbuf, acc, sas, sad, sbs, sbd, sca, scb):
    cid = lax.axis_index("c")
    sid = lax.axis_index("s")

    @pl.when(cid == 0)
    def _():
        _zero_acc(zbuf, acc, sid, nj * 16)
        pltpu.sync_copy(mexp, mexp_v)
        plsc.subcore_barrier()

        mvs = [mexp_v[pl.ds(j * 16, 16)] for j in range(nj)]
        start = mk_start(sid)

        def _compute(srows, drows, msg):
            def _edge(ee, c2):
                for j in range(nj):
                    a = srows[ee, pl.ds(hoff + j * 16, 16)]
                    b = drows[ee, pl.ds(j * 16, 16)]
                    t = a + b
                    t = jnp.maximum(t, 0.2 * t) - mvs[j]
                    ex = jnp.exp(t)
                    hv = srows[ee, pl.ds(j * 16, 16)]
                    msg[ee, pl.ds(j * 16, 16)] = hv * ex
                    if hoff == 64:
                        msg[ee, pl.ds(64 + j * 16, 16)] = ex
                return c2

            lax.fori_loop(0, chunk, _edge, 0)

        def _pair(g, c):
            row = start + 2 * g

            @pl.when(g > 0)
            def _():
                pltpu.make_async_copy(msga, acc.at[didxa], sca).wait()

            pltpu.sync_copy(sidx.at[row], sidxa)
            pltpu.sync_copy(didx.at[row], didxa)
            ga_s = pltpu.make_async_copy(stab.at[sidxa], sra, sas)
            ga_d = pltpu.make_async_copy(adtab.at[didxa], dra, sad)
            ga_s.start()
            ga_d.start()

            @pl.when(g > 0)
            def _():
                pltpu.make_async_copy(msgb, acc.at[didxb], scb).wait()

            pltpu.sync_copy(sidx.at[row + 1], sidxb)
            pltpu.sync_copy(didx.at[row + 1], didxb)
            gb_s = pltpu.make_async_copy(stab.at[sidxb], srb, sbs)
            gb_d = pltpu.make_async_copy(adtab.at[didxb], drb, sbd)
            gb_s.start()
            gb_d.start()
            ga_s.wait()
            ga_d.wait()
            _compute(sra, dra, msga)
            da = pltpu.make_async_copy(msga, acc.at[didxa], sca)
            da.start(add=True)
            gb_s.wait()
            gb_d.wait()
            _compute(srb, drb, msgb)
            db = pltpu.make_async_copy(msgb, acc.at[didxb], scb)
            db.start(add=True)
            return c

        lax.fori_loop(0, npairs, _pair, 0)
        pltpu.make_async_copy(msga, acc.at[didxa], sca).wait()
        pltpu.make_async_copy(msgb, acc.at[didxb], scb).wait()

        @pl.when(tail_cond(sid))
        def _tail():
            row = start + 2 * npairs
            pltpu.sync_copy(sidx.at[row], sidxa)
            pltpu.sync_copy(didx.at[row], didxa)
            ga_s = pltpu.make_async_copy(stab.at[sidxa], sra, sas)
            ga_d = pltpu.make_async_copy(adtab.at[didxa], dra, sad)
            ga_s.start()
            ga_d.start()
            ga_s.wait()
            ga_d.wait()
            _compute(sra, dra, msga)
            da = pltpu.make_async_copy(msga, acc.at[didxa], sca)
            da.start(add=True)
            da.wait()

        plsc.subcore_barrier()
        _publish_acc(acc, out, cid, sid)


def _make_edge_pass(hoff, nj, chunk, npairs, mk_start, tail_cond, sw, dw, aw):
    body = functools.partial(_edge_pass_body, hoff, nj, chunk, npairs,
                             mk_start, tail_cond)
    return functools.partial(
        pl.kernel,
        out_type=jax.ShapeDtypeStruct((N, aw), jnp.float32),
        mesh=_SC_MESH,
        compiler_params=_SC_PARAMS,
        scratch_types=[
            pltpu.VMEM((chunk,), jnp.int32),
            pltpu.VMEM((chunk,), jnp.int32),
            pltpu.VMEM((chunk,), jnp.int32),
            pltpu.VMEM((chunk,), jnp.int32),
            pltpu.VMEM((chunk, sw), jnp.float32),
            pltpu.VMEM((chunk, dw), jnp.float32),
            pltpu.VMEM((chunk, sw), jnp.float32),
            pltpu.VMEM((chunk, dw), jnp.float32),
            pltpu.VMEM((chunk, aw), jnp.float32),
            pltpu.VMEM((chunk, aw), jnp.float32),
            pltpu.VMEM((nj * 16,), jnp.float32),
            pltpu.VMEM((ZR, aw), jnp.float32),
            pltpu.VMEM_SHARED((N, aw), jnp.float32),
            pltpu.SemaphoreType.DMA,
            pltpu.SemaphoreType.DMA,
            pltpu.SemaphoreType.DMA,
            pltpu.SemaphoreType.DMA,
            pltpu.SemaphoreType.DMA,
            pltpu.SemaphoreType.DMA,
        ],
    )(body)


_sc1 = _make_edge_pass(
    64, 4, CHUNK1, (EROWS1 // NSUB) // 2,
    lambda sid: sid * (EROWS1 // NSUB), lambda sid: sid >= 0,
    SW1, DW1, AW1)
_sc2 = _make_edge_pass(
    48, 3, CHUNK2, 156,
    lambda sid: sid * 313 - jnp.maximum(sid - 8, 0), lambda sid: sid < 8,
    SW2, DW2, AW2)


def _mid_body(acc_ref, stab_ref, adtab_ref, mexp_ref, b1_ref, w2_ref,
              a2s_ref, a2d_ref, stab2_ref, adtab2_ref, m2_ref):
    i = pl.program_id(0)
    acc = acc_ref[...]
    h1 = stab_ref[:, 0:64]
    as_e = stab_ref[:, 64:128]
    ad_e = adtab_ref[...]
    t = as_e + ad_e
    exs = jnp.exp(jnp.maximum(t, 0.2 * t) - mexp_ref[...])
    num = acc[:, 0:64] + exs * h1
    den = acc[:, 64:128] + exs + 1e-16
    o1 = num / den + b1_ref[...]
    hmid = jnp.where(o1 > 0, o1, jnp.exp(jnp.minimum(o1, 0.0)) - 1.0)
    h2 = jnp.dot(hmid, w2_ref[...], preferred_element_type=jnp.float32)
    as2 = jnp.dot(h2, a2s_ref[...], preferred_element_type=jnp.float32)
    ad2 = jnp.dot(h2, a2d_ref[...], preferred_element_type=jnp.float32)
    ones = jnp.ones((RB, 1), jnp.float32)
    stab2_ref[...] = jnp.concatenate(
        [h2, ones, jnp.zeros((RB, 7), jnp.float32),
         jnp.broadcast_to(as2, (RB, 48))], axis=1)
    adtab2_ref[...] = jnp.broadcast_to(ad2, (RB, DW2))
    bm = jnp.concatenate(
        [jnp.max(as2, axis=0, keepdims=True),
         jnp.max(ad2, axis=0, keepdims=True),
         jnp.zeros((1, 14), jnp.float32)], axis=1)

    @pl.when(i == 0)
    def _():
        m2_ref[...] = bm

    @pl.when(i > 0)
    def _():
        m2_ref[...] = jnp.maximum(m2_ref[...], bm)


def _mid(acc1, stab, adtab, mexp1, b1, W2, A2s, A2d):
    return pl.pallas_call(
        _mid_body,
        grid=(N // RB,),
        in_specs=[
            pl.BlockSpec((RB, AW1), lambda i: (i, 0)),
            pl.BlockSpec((RB, SW1), lambda i: (i, 0)),
            pl.BlockSpec((RB, DW1), lambda i: (i, 0)),
            pl.BlockSpec((1, 64), lambda i: (0, 0)),
            pl.BlockSpec((1, 64), lambda i: (0, 0)),
            pl.BlockSpec((64, NC), lambda i: (0, 0)),
            pl.BlockSpec((NC, 1), lambda i: (0, 0)),
            pl.BlockSpec((NC, 1), lambda i: (0, 0)),
        ],
        out_specs=[
            pl.BlockSpec((RB, SW2), lambda i: (i, 0)),
            pl.BlockSpec((RB, DW2), lambda i: (i, 0)),
            pl.BlockSpec((1, 16), lambda i: (0, 0)),
        ],
        out_shape=[
            jax.ShapeDtypeStruct((N, SW2), jnp.float32),
            jax.ShapeDtypeStruct((N, DW2), jnp.float32),
            jax.ShapeDtypeStruct((1, 16), jnp.float32),
        ],
    )(acc1, stab, adtab, mexp1, b1, W2, A2s, A2d)


def _fin_body(acc_ref, stab2_ref, adtab2_ref, m2_ref, b2_ref, out_ref):
    acc = acc_ref[...]
    h2 = stab2_ref[:, 0:40]
    as2 = stab2_ref[:, 48:49]
    ad2 = adtab2_ref[:, 0:1]
    m2 = m2_ref[0, 0] + m2_ref[0, 1]
    m2 = jnp.maximum(m2, 0.2 * m2)
    t = as2 + ad2
    ex = jnp.exp(jnp.maximum(t, 0.2 * t) - m2)
    num = acc[:, 0:40] + ex * h2
    den = acc[:, 40:41] + ex + 1e-16
    o2 = num / den + b2_ref[...]
    mx = jnp.max(o2, axis=1, keepdims=True)
    z = o2 - mx
    lse = jnp.log(jnp.sum(jnp.exp(z), axis=1, keepdims=True))
    out_ref[...] = z - lse


def _fin(acc2, stab2, adtab2, m2, b2):
    return pl.pallas_call(
        _fin_body,
        grid=(N // RB,),
        in_specs=[
            pl.BlockSpec((RB, AW2), lambda i: (i, 0)),
            pl.BlockSpec((RB, SW2), lambda i: (i, 0)),
            pl.BlockSpec((RB, DW2), lambda i: (i, 0)),
            pl.BlockSpec((1, 16), lambda i: (0, 0)),
            pl.BlockSpec((1, NC), lambda i: (0, 0)),
        ],
        out_specs=pl.BlockSpec((RB, NC), lambda i: (i, 0)),
        out_shape=jax.ShapeDtypeStruct((N, NC), jnp.float32),
    )(acc2, stab2, adtab2, m2, b2)


def kernel(x, edge_index, W1, a_src1, a_dst1, b1, W2, a_src2, a_dst2, b2):
    src_i = edge_index[0].astype(jnp.int32)
    dst_i = edge_index[1].astype(jnp.int32)
    sidx1 = src_i.reshape(EROWS1, CHUNK1)
    didx1 = dst_i.reshape(EROWS1, CHUNK1)
    sidx2v = src_i.reshape(EROWS2, CHUNK2)
    didx2v = dst_i.reshape(EROWS2, CHUNK2)

    r8 = np.zeros((H1, H1 * C1), np.float32)
    for hh in range(H1):
        r8[hh, hh * C1:(hh + 1) * C1] = 1.0
    R8 = jnp.asarray(r8)
    Asrc_exp = _expand_a(a_src1, H1, C1) @ R8
    Adst_exp = _expand_a(a_dst1, H1, C1) @ R8

    stab, adtab, m1 = _prep1(x, W1, Asrc_exp, Adst_exp)
    ms = m1[0, 0:64] + m1[0, 64:128]
    mexp1 = jnp.maximum(ms, 0.2 * ms)

    acc1 = _sc1(stab, adtab, sidx1, didx1, mexp1)

    stab2, adtab2, m2 = _mid(acc1, stab, adtab, mexp1.reshape(1, 64),
                             b1.reshape(1, -1), W2,
                             a_src2.reshape(-1, 1), a_dst2.reshape(-1, 1))
    s2 = m2[0, 0] + m2[0, 1]
    M2 = jnp.maximum(s2, 0.2 * s2)
    mexp2 = jnp.full((48,), M2, jnp.float32)

    acc2 = _sc2(stab2, adtab2, sidx2v, didx2v, mexp2)

    return _fin(acc2, stab2, adtab2, m2, b2.reshape(1, -1))


def _expand_a(a, H, C):
    out = jnp.zeros((H * C, H), jnp.float32)
    idx = jnp.arange(H * C)
    return out.at[idx, idx // C].set(a.reshape(-1))
